# baseline (speedup 1.0000x reference)
import jax
import jax.numpy as jnp
from jax.experimental import pallas as pl


def _identity_pallas(x):
    def body(x_ref, o_ref):
        o_ref[...] = x_ref[...]
    return pl.pallas_call(body, out_shape=jax.ShapeDtypeStruct(x.shape, x.dtype))(x)


def _sage(x, src, dst, Wl, bl, Wr):
    n = x.shape[0]
    agg = jax.ops.segment_sum(x[src], dst, num_segments=n)
    cnt = jax.ops.segment_sum(jnp.ones((src.shape[0],), x.dtype), dst, num_segments=n)
    mean = agg / jnp.clip(cnt, 1.0)[:, None]
    return mean @ Wl + bl + x @ Wr


def kernel(freq, edge_index, edge_weight, Wl0, bl0, Wr0, Wl1, bl1, Wr1, Wl2, bl2, Wr2, pool_w, W1, b1, W2, b2):
    Bb, Nn, Dd = freq.shape
    x = freq.reshape(Bb * Nn, Dd)
    offs = (jnp.arange(Bb, dtype=edge_index.dtype) * Nn)[:, None, None]
    ei = edge_index + offs
    src = ei[:, 0, :].reshape(-1)
    dst = ei[:, 1, :].reshape(-1)
    x = jax.nn.relu(_sage(x, src, dst, Wl0, bl0, Wr0))
    x = jax.nn.relu(_sage(x, src, dst, Wl1, bl1, Wr1))
    x = _sage(x, src, dst, Wl2, bl2, Wr2)
    score = jnp.tanh((x @ pool_w) / jnp.linalg.norm(pool_w))
    sb = score.reshape(Bb, Nn)
    k = -(-Nn // 2)
    vals, idx = jax.lax.top_k(sb, k)
    xb = x.reshape(Bb, Nn, -1)
    xsel = jnp.take_along_axis(xb, idx[:, :, None], axis=1) * vals[:, :, None]
    pooled = xsel.mean(axis=1)
    h = jax.nn.relu(pooled @ W1 + b1)
    h = _identity_pallas(h)
    return h @ W2 + b2


# speedup vs baseline: 1.9360x; 1.9360x over previous
import functools

import jax
import jax.numpy as jnp
from jax import lax
from jax.experimental import pallas as pl
from jax.experimental.pallas import tpu as pltpu
from jax.experimental.pallas import tpu_sc as plsc

B, N, D, E = 4, 10000, 128, 160000
NT = B * N
ET = B * E
RPB = E // 128
RPW = 160
RPG = 8 * RPW
ROWS = B * RPG
HALF = 20480
NPAD = 2 * HALF
ACC_ROWS = HALF + 128
DUMP = HALF
CHALF = HALF + 16
K = NT // B // 2
F = 64

def _prep_body(ei4, srcg, dl, inv, sslab, dslab, cntv, tmpv, invv, stage):
    c = lax.axis_index("c")
    s = lax.axis_index("s")
    ones16 = jnp.ones((16,), jnp.float32)

    def zero_cnt(i, _):
        cntv[pl.ds(i * 16, 16)] = jnp.zeros((16,), jnp.float32)
        return 0

    lax.fori_loop(0, CHALF // 16, zero_cnt, 0)

    for half in range(2):
        row_base = (2 * s + half) * RPW
        b = row_base // RPG
        rib = row_base - b * RPG
        pltpu.sync_copy(ei4.at[b, 0, pl.ds(rib, RPW)], sslab)
        pltpu.sync_copy(ei4.at[b, 1, pl.ds(rib, RPW)], dslab)
        boff = b * N

        def row_body(q, _):
            realrow = rib + q < RPB
            for ci in range(8):
                sl = sslab[q, pl.ds(16 * ci, 16)]
                dg = dslab[q, pl.ds(16 * ci, 16)] + boff
                sg = jnp.where(realrow, sl + boff, 0)
                d0 = jnp.where(realrow & (dg < HALF), dg, DUMP)
                d1 = jnp.where(realrow & (dg >= HALF), dg - HALF, DUMP)
                dloc = dg - c * HALF
                okc = realrow & (dloc >= 0) & (dloc < HALF)
                dcnt = jnp.where(okc, dloc, HALF)
                plsc.addupdate_scatter(cntv, [dcnt], ones16)
                sslab[q, pl.ds(16 * ci, 16)] = sg
                dslab[q, pl.ds(16 * ci, 16)] = jnp.where(c == 0, d0, d1)
            return 0

        lax.fori_loop(0, RPW, row_body, 0)

        @pl.when(c == 0)
        def _():
            pltpu.sync_copy(sslab, srcg.at[pl.ds(row_base, RPW)])

        pltpu.sync_copy(dslab, dl.at[c, pl.ds(row_base, RPW)])

    pltpu.sync_copy(cntv.at[pl.ds(0, HALF)], stage.at[s])
    plsc.subcore_barrier()
    for t in range(16):
        pltpu.sync_copy(stage.at[t, pl.ds(s * 1280, 1280)], tmpv.at[t])

    def red(jj, _):
        acc = tmpv[0, pl.ds(16 * jj, 16)]
        for t in range(1, 16):
            acc = acc + tmpv[t, pl.ds(16 * jj, 16)]
        invv[pl.ds(16 * jj, 16)] = 1.0 / jnp.maximum(acc, 1.0)
        return 0

    lax.fori_loop(0, 80, red, 0)
    pltpu.sync_copy(invv, inv.at[pl.ds(c * HALF + s * 1280, 1280)])


def _seg_body(tblA, tblB, srcg, dl, aggA, aggB, sidx, didx, stage, zbuf, acc, gsem):
    c = lax.axis_index("c")
    s = lax.axis_index("s")

    def zz(i, _):
        for k in range(4):
            zbuf[i, pl.ds(16 * k, 16)] = jnp.zeros((16,), jnp.float32)
        return 0

    lax.fori_loop(0, 128, zz, 0)

    for p, (tbl, out) in enumerate(((tblA, aggA), (tblB, aggB))):
        zb = s * (ACC_ROWS // 16)
        for k in range(10):
            pltpu.sync_copy(zbuf, acc.at[pl.ds(zb + 128 * k, 128)])
        pltpu.sync_copy(zbuf.at[pl.ds(0, ACC_ROWS // 16 - 1280)],
                        acc.at[pl.ds(zb + 1280, ACC_ROWS // 16 - 1280)])
        plsc.subcore_barrier()

        def it_body(it, _):
            r0 = (ROWS // 16) * s + 8 * it
            pltpu.sync_copy(srcg.at[pl.ds(r0, 8)], sidx)
            pltpu.sync_copy(dl.at[c, pl.ds(r0, 8)], didx)
            for h in range(2):
                descs = [
                    pltpu.async_copy(tbl.at[sidx.at[4 * h + j]],
                                     stage.at[pl.ds(128 * j, 128)], gsem)
                    for j in range(4)
                ]
                for j in range(4):
                    descs[j].wait()
                for j in range(4):
                    pltpu.sync_copy(stage.at[pl.ds(128 * j, 128)],
                                    acc.at[didx.at[4 * h + j]], add=True)
            return 0

        lax.fori_loop(0, ROWS // 16 // 8, it_body, 0)
        plsc.subcore_barrier()
        pltpu.sync_copy(acc.at[pl.ds(s * 1280, 1280)],
                        out.at[pl.ds(c * HALF + s * 1280, 1280)])
        plsc.subcore_barrier()


@functools.lru_cache(maxsize=None)
def _sc_kernels():
    mesh = plsc.VectorSubcoreMesh(core_axis_name="c", subcore_axis_name="s")
    prep = pl.kernel(
        _prep_body,
        out_type=(
            jax.ShapeDtypeStruct((ROWS, 128), jnp.int32),
            jax.ShapeDtypeStruct((2, ROWS, 128), jnp.int32),
            jax.ShapeDtypeStruct((NPAD,), jnp.float32),
        ),
        mesh=mesh,
        scratch_types=[
            pltpu.VMEM((RPW, 128), jnp.int32),
            pltpu.VMEM((RPW, 128), jnp.int32),
            pltpu.VMEM((CHALF,), jnp.float32),
            pltpu.VMEM((16, 1280), jnp.float32),
            pltpu.VMEM((1280,), jnp.float32),
            pltpu.VMEM_SHARED((16, HALF), jnp.float32),
        ],
        compiler_params=pltpu.CompilerParams(
            needs_layout_passes=False, use_tc_tiling_on_sc=False),
    )
    seg = pl.kernel(
        _seg_body,
        out_type=(
            jax.ShapeDtypeStruct((NPAD, F), jnp.float32),
            jax.ShapeDtypeStruct((NPAD, F), jnp.float32),
        ),
        mesh=mesh,
        scratch_types=[
            pltpu.VMEM((8, 128), jnp.int32),
            pltpu.VMEM((8, 128), jnp.int32),
            pltpu.VMEM((512, F), jnp.float32),
            pltpu.VMEM((128, F), jnp.float32),
            pltpu.VMEM_SHARED((ACC_ROWS, F), jnp.float32),
            pltpu.SemaphoreType.DMA,
        ],
        compiler_params=pltpu.CompilerParams(
            needs_layout_passes=False, use_tc_tiling_on_sc=False),
    )
    return prep, seg


def _layer_body(relu, aggA, aggB, hA, hB, inv, Wl, Wr, bl, oA, oB):
    mean = jnp.concatenate([aggA[...], aggB[...]], axis=1) * inv[...]
    x = jnp.concatenate([hA[...], hB[...]], axis=1)
    y = (jnp.dot(mean, Wl[...], preferred_element_type=jnp.float32)
         + jnp.dot(x, Wr[...], preferred_element_type=jnp.float32) + bl[...])
    if relu:
        y = jnp.maximum(y, 0.0)
    oA[...] = y[:, :F]
    oB[...] = y[:, F:]


def _layer(aggA, aggB, hA, hB, inv2, Wl, Wr, bl, relu):
    R = 2560
    grid = NPAD // R
    io = lambda i: (i, 0)
    w0 = lambda i: (0, 0)
    return pl.pallas_call(
        functools.partial(_layer_body, relu),
        grid=(grid,),
        in_specs=[
            pl.BlockSpec((R, F), io), pl.BlockSpec((R, F), io),
            pl.BlockSpec((R, F), io), pl.BlockSpec((R, F), io),
            pl.BlockSpec((R, 1), io),
            pl.BlockSpec((128, 128), w0), pl.BlockSpec((128, 128), w0),
            pl.BlockSpec((1, 128), w0),
        ],
        out_specs=[pl.BlockSpec((R, F), io), pl.BlockSpec((R, F), io)],
        out_shape=[jax.ShapeDtypeStruct((NPAD, F), jnp.float32)] * 2,
    )(aggA, aggB, hA, hB, inv2, Wl, Wr, bl.reshape(1, 128))


def _pool_body(hA, hB, pw, pooled):
    x = jnp.concatenate([hA[...], hB[...]], axis=1)
    pwv = pw[...]
    pwn = pwv * lax.rsqrt(jnp.sum(pwv * pwv))
    sc = lax.dot_general(x, pwn, (((1,), (1,)), ((), ())),
                         preferred_element_type=jnp.float32)
    kb = lax.bitcast_convert_type(sc, jnp.int32)
    k = kb ^ ((kb >> 31) & jnp.int32(0x7FFFFFFF))
    ku = lax.bitcast_convert_type(k ^ jnp.int32(-2147483648), jnp.uint32)
    top = jnp.uint32(2147483648)

    def step(j, t):
        cand = t | lax.shift_right_logical(top, jnp.uint32(j))
        cnt = jnp.sum((ku >= cand).astype(jnp.int32))
        return jnp.where(cnt >= K, cand, t)

    t = lax.fori_loop(0, 32, step, jnp.uint32(0))
    w = jnp.tanh(sc) * (ku >= t).astype(jnp.float32)
    pr = lax.dot_general(w, x, (((0,), (0,)), ((), ())),
                         preferred_element_type=jnp.float32)
    pooled[pl.ds(pl.program_id(0), 1), :] = pr * (1.0 / K)


def _pool(hA, hB, pool_w):
    return pl.pallas_call(
        _pool_body,
        grid=(B,),
        in_specs=[
            pl.BlockSpec((N, F), lambda b: (b, 0)),
            pl.BlockSpec((N, F), lambda b: (b, 0)),
            pl.BlockSpec((1, 128), lambda b: (0, 0)),
        ],
        out_specs=pl.BlockSpec((B, 128), lambda b: (0, 0)),
        out_shape=jax.ShapeDtypeStruct((B, 128), jnp.float32),
    )(hA, hB, pool_w.reshape(1, 128))


def _mlp_body(p, W1, b1, W2, b2, o):
    h = jnp.maximum(
        jnp.dot(p[...], W1[...], preferred_element_type=jnp.float32) + b1[...],
        0.0)
    o[...] = jnp.dot(h, W2[...], preferred_element_type=jnp.float32) + b2[...]


def _mlp(p, W1, b1, W2, b2):
    fs = lambda *shape: pl.BlockSpec(shape, lambda: tuple(0 for _ in shape))
    return pl.pallas_call(
        _mlp_body,
        in_specs=[fs(B, 128), fs(128, 256), fs(1, 256), fs(256, 10), fs(1, 10)],
        out_specs=fs(B, 10),
        out_shape=jax.ShapeDtypeStruct((B, 10), jnp.float32),
    )(p, W1, b1.reshape(1, 256), W2, b2.reshape(1, 10))


def kernel(freq, edge_index, edge_weight, Wl0, bl0, Wr0, Wl1, bl1, Wr1,
           Wl2, bl2, Wr2, pool_w, W1, b1, W2, b2):
    ei4 = jnp.pad(edge_index.reshape(B, 2, RPB, 128),
                  ((0, 0), (0, 0), (0, RPG - RPB), (0, 0)))
    _prep, _seg = _sc_kernels()
    srcg, dlp, inv = _prep(ei4)
    inv2 = inv.reshape(NPAD, 1)
    x = freq.reshape(NT, D)
    xp = jnp.pad(x, ((0, NPAD - NT), (0, 0)))
    hA, hB = xp[:, :F], xp[:, F:]
    for i, (Wl, bl, Wr) in enumerate(((Wl0, bl0, Wr0), (Wl1, bl1, Wr1),
                                      (Wl2, bl2, Wr2))):
        aggA, aggB = _seg(hA, hB, srcg, dlp)
        hA, hB = _layer(aggA, aggB, hA, hB, inv2, Wl, Wr, bl, relu=(i < 2))
    pooled = _pool(hA[:NT], hB[:NT], pool_w)
    return _mlp(pooled, W1, b1, W2, b2)


# speedup vs baseline: 3.0589x; 1.5800x over previous
import functools

import jax
import jax.numpy as jnp
from jax import lax
from jax.experimental import pallas as pl
from jax.experimental.pallas import tpu as pltpu
from jax.experimental.pallas import tpu_sc as plsc

B, N, D, E = 4, 10000, 128, 160000
NT = B * N
ET = B * E
RPB = E // 128
RPW = 160
RPG = 8 * RPW
ROWS = B * RPG
HALF = 20480
NPAD = 2 * HALF
ACC_ROWS = NPAD + 128
CHALF = HALF + 16
K = NT // B // 2
F = 32

def _prep_body(ei4, srcg, dl, inv, sslab, dslab, cntv, tmpv, invv, stage):
    c = lax.axis_index("c")
    s = lax.axis_index("s")
    ones16 = jnp.ones((16,), jnp.float32)

    def zero_cnt(i, _):
        cntv[pl.ds(i * 16, 16)] = jnp.zeros((16,), jnp.float32)
        return 0

    lax.fori_loop(0, CHALF // 16, zero_cnt, 0)

    for half in range(2):
        row_base = (2 * s + half) * RPW
        b = row_base // RPG
        rib = row_base - b * RPG
        pltpu.sync_copy(ei4.at[b, 0, pl.ds(rib, RPW)], sslab)
        pltpu.sync_copy(ei4.at[b, 1, pl.ds(rib, RPW)], dslab)
        boff = b * N

        def row_body(q, _):
            realrow = rib + q < RPB
            for ci in range(8):
                sl = sslab[q, pl.ds(16 * ci, 16)]
                dg = dslab[q, pl.ds(16 * ci, 16)] + boff
                sg = jnp.where(realrow, sl + boff, 0)
                dgp = jnp.where(realrow, dg, NPAD)
                dloc = dg - c * HALF
                okc = realrow & (dloc >= 0) & (dloc < HALF)
                dcnt = jnp.where(okc, dloc, HALF)
                plsc.addupdate_scatter(cntv, [dcnt], ones16)
                sslab[q, pl.ds(16 * ci, 16)] = sg
                dslab[q, pl.ds(16 * ci, 16)] = dgp
            return 0

        lax.fori_loop(0, RPW, row_body, 0)

        @pl.when(c == 0)
        def _():
            pltpu.sync_copy(sslab, srcg.at[pl.ds(row_base, RPW)])
            pltpu.sync_copy(dslab, dl.at[pl.ds(row_base, RPW)])

    pltpu.sync_copy(cntv.at[pl.ds(0, HALF)], stage.at[s])
    plsc.subcore_barrier()
    for t in range(16):
        pltpu.sync_copy(stage.at[t, pl.ds(s * 1280, 1280)], tmpv.at[t])

    def red(jj, _):
        acc = tmpv[0, pl.ds(16 * jj, 16)]
        for t in range(1, 16):
            acc = acc + tmpv[t, pl.ds(16 * jj, 16)]
        invv[pl.ds(16 * jj, 16)] = 1.0 / jnp.maximum(acc, 1.0)
        return 0

    lax.fori_loop(0, 80, red, 0)
    pltpu.sync_copy(invv, inv.at[pl.ds(c * HALF + s * 1280, 1280)])


def _seg_body(t0, t1, t2, t3, srcg, dg, out, sidx, didx, stage, zbuf, acc,
              gsems, ssems):
    c = lax.axis_index("c")
    s = lax.axis_index("s")

    def zz(i, _):
        for k in range(2):
            zbuf[i, pl.ds(16 * k, 16)] = jnp.zeros((16,), jnp.float32)
        return 0

    lax.fori_loop(0, 128, zz, 0)

    n_it = (ROWS // 2) // 16 // 8

    for p, tbl in enumerate((t0, t1, t2, t3)):
        zb = s * (ACC_ROWS // 16)
        for k in range(20):
            pltpu.sync_copy(zbuf, acc.at[pl.ds(zb + 128 * k, 128)])
        pltpu.sync_copy(zbuf.at[pl.ds(0, ACC_ROWS // 16 - 2560)],
                        acc.at[pl.ds(zb + 2560, ACC_ROWS // 16 - 2560)])
        plsc.subcore_barrier()

        def it_body(it, _):
            @pl.when(it > 0)
            def _():
                for j in range(8):
                    pltpu.make_async_copy(stage.at[pl.ds(128 * j, 128)],
                                          acc.at[didx.at[j]],
                                          ssems.at[j]).wait()

            r0 = c * (ROWS // 2) + s * (8 * n_it) + 8 * it
            pltpu.sync_copy(srcg.at[pl.ds(r0, 8)], sidx)
            pltpu.sync_copy(dg.at[pl.ds(r0, 8)], didx)
            descs = [
                pltpu.async_copy(tbl.at[sidx.at[j]],
                                 stage.at[pl.ds(128 * j, 128)], gsems.at[j])
                for j in range(8)
            ]
            for j in range(8):
                descs[j].wait()
                pltpu.async_copy(stage.at[pl.ds(128 * j, 128)],
                                 acc.at[didx.at[j]], ssems.at[j], add=True)
            return 0

        lax.fori_loop(0, n_it, it_body, 0)
        for j in range(8):
            pltpu.make_async_copy(stage.at[pl.ds(128 * j, 128)],
                                  acc.at[didx.at[j]], ssems.at[j]).wait()
        plsc.subcore_barrier()
        pltpu.sync_copy(acc.at[pl.ds(s * 2560, 2560)],
                        out.at[c, p, pl.ds(s * 2560, 2560)])
        plsc.subcore_barrier()


@functools.lru_cache(maxsize=None)
def _sc_kernels():
    mesh = plsc.VectorSubcoreMesh(core_axis_name="c", subcore_axis_name="s")
    prep = pl.kernel(
        _prep_body,
        out_type=(
            jax.ShapeDtypeStruct((ROWS, 128), jnp.int32),
            jax.ShapeDtypeStruct((ROWS, 128), jnp.int32),
            jax.ShapeDtypeStruct((NPAD,), jnp.float32),
        ),
        mesh=mesh,
        scratch_types=[
            pltpu.VMEM((RPW, 128), jnp.int32),
            pltpu.VMEM((RPW, 128), jnp.int32),
            pltpu.VMEM((CHALF,), jnp.float32),
            pltpu.VMEM((16, 1280), jnp.float32),
            pltpu.VMEM((1280,), jnp.float32),
            pltpu.VMEM_SHARED((16, HALF), jnp.float32),
        ],
        compiler_params=pltpu.CompilerParams(
            needs_layout_passes=False, use_tc_tiling_on_sc=False),
    )
    seg = pl.kernel(
        _seg_body,
        out_type=jax.ShapeDtypeStruct((2, 4, NPAD, F), jnp.float32),
        mesh=mesh,
        scratch_types=[
            pltpu.VMEM((8, 128), jnp.int32),
            pltpu.VMEM((8, 128), jnp.int32),
            pltpu.VMEM((1024, F), jnp.float32),
            pltpu.VMEM((128, F), jnp.float32),
            pltpu.VMEM_SHARED((ACC_ROWS, F), jnp.float32),
            pltpu.SemaphoreType.DMA((8,)),
            pltpu.SemaphoreType.DMA((8,)),
        ],
        compiler_params=pltpu.CompilerParams(
            needs_layout_passes=False, use_tc_tiling_on_sc=False),
    )
    return prep, seg


def _layer_body(relu, agg, h0, h1, h2, h3, inv, Wl, Wr, bl, o0, o1, o2, o3):
    a = agg[...]
    ap = a[0] + a[1]
    mean = jnp.concatenate([ap[0], ap[1], ap[2], ap[3]], axis=1) * inv[...]
    x = jnp.concatenate([h0[...], h1[...], h2[...], h3[...]], axis=1)
    y = (jnp.dot(mean, Wl[...], preferred_element_type=jnp.float32)
         + jnp.dot(x, Wr[...], preferred_element_type=jnp.float32) + bl[...])
    if relu:
        y = jnp.maximum(y, 0.0)
    o0[...] = y[:, 0 * F:1 * F]
    o1[...] = y[:, 1 * F:2 * F]
    o2[...] = y[:, 2 * F:3 * F]
    o3[...] = y[:, 3 * F:4 * F]


def _layer(agg, hs, inv2, Wl, Wr, bl, relu):
    R = 2560
    grid = NPAD // R
    io = lambda i: (i, 0)
    w0 = lambda i: (0, 0)
    return pl.pallas_call(
        functools.partial(_layer_body, relu),
        grid=(grid,),
        in_specs=[
            pl.BlockSpec((2, 4, R, F), lambda i: (0, 0, i, 0)),
            pl.BlockSpec((R, F), io), pl.BlockSpec((R, F), io),
            pl.BlockSpec((R, F), io), pl.BlockSpec((R, F), io),
            pl.BlockSpec((R, 1), io),
            pl.BlockSpec((128, 128), w0), pl.BlockSpec((128, 128), w0),
            pl.BlockSpec((1, 128), w0),
        ],
        out_specs=[pl.BlockSpec((R, F), io)] * 4,
        out_shape=[jax.ShapeDtypeStruct((NPAD, F), jnp.float32)] * 4,
    )(agg, *hs, inv2, Wl, Wr, bl.reshape(1, 128))


def _pool_body(h0, h1, h2, h3, pw, pooled):
    x = jnp.concatenate([h0[...], h1[...], h2[...], h3[...]], axis=1)
    pwv = pw[...]
    pwn = pwv * lax.rsqrt(jnp.sum(pwv * pwv))
    sc = lax.dot_general(x, pwn, (((1,), (1,)), ((), ())),
                         preferred_element_type=jnp.float32)
    kb = lax.bitcast_convert_type(sc, jnp.int32)
    k = kb ^ ((kb >> 31) & jnp.int32(0x7FFFFFFF))
    ku = lax.bitcast_convert_type(k ^ jnp.int32(-2147483648), jnp.uint32)
    top = jnp.uint32(2147483648)

    def step(j, t):
        cand = t | lax.shift_right_logical(top, jnp.uint32(j))
        cnt = jnp.sum((ku >= cand).astype(jnp.int32))
        return jnp.where(cnt >= K, cand, t)

    t = lax.fori_loop(0, 32, step, jnp.uint32(0))
    w = jnp.tanh(sc) * (ku >= t).astype(jnp.float32)
    pr = lax.dot_general(w, x, (((0,), (0,)), ((), ())),
                         preferred_element_type=jnp.float32)
    pooled[pl.ds(pl.program_id(0), 1), :] = pr * (1.0 / K)


def _pool(hs, pool_w):
    return pl.pallas_call(
        _pool_body,
        grid=(B,),
        in_specs=[pl.BlockSpec((N, F), lambda b: (b, 0))] * 4
        + [pl.BlockSpec((1, 128), lambda b: (0, 0))],
        out_specs=pl.BlockSpec((B, 128), lambda b: (0, 0)),
        out_shape=jax.ShapeDtypeStruct((B, 128), jnp.float32),
    )(*hs, pool_w.reshape(1, 128))


def _mlp_body(p, W1, b1, W2, b2, o):
    h = jnp.maximum(
        jnp.dot(p[...], W1[...], preferred_element_type=jnp.float32) + b1[...],
        0.0)
    o[...] = jnp.dot(h, W2[...], preferred_element_type=jnp.float32) + b2[...]


def _mlp(p, W1, b1, W2, b2):
    fs = lambda *shape: pl.BlockSpec(shape, lambda: tuple(0 for _ in shape))
    return pl.pallas_call(
        _mlp_body,
        in_specs=[fs(B, 128), fs(128, 256), fs(1, 256), fs(256, 10), fs(1, 10)],
        out_specs=fs(B, 10),
        out_shape=jax.ShapeDtypeStruct((B, 10), jnp.float32),
    )(p, W1, b1.reshape(1, 256), W2, b2.reshape(1, 10))


def kernel(freq, edge_index, edge_weight, Wl0, bl0, Wr0, Wl1, bl1, Wr1,
           Wl2, bl2, Wr2, pool_w, W1, b1, W2, b2):
    ei4 = jnp.pad(edge_index.reshape(B, 2, RPB, 128),
                  ((0, 0), (0, 0), (0, RPG - RPB), (0, 0)))
    _prep, _seg = _sc_kernels()
    srcg, dgp, inv = _prep(ei4)
    inv2 = inv.reshape(NPAD, 1)
    x = freq.reshape(NT, D)
    xp = jnp.pad(x, ((0, NPAD - NT), (0, 0)))
    hs = [xp[:, i * F:(i + 1) * F] for i in range(4)]
    for i, (Wl, bl, Wr) in enumerate(((Wl0, bl0, Wr0), (Wl1, bl1, Wr1),
                                      (Wl2, bl2, Wr2))):
        agg = _seg(*hs, srcg, dgp)
        hs = _layer(agg, hs, inv2, Wl, Wr, bl, relu=(i < 2))
    pooled = _pool([h[:NT] for h in hs], pool_w)
    return _mlp(pooled, W1, b1, W2, b2)


# speedup vs baseline: 3.2767x; 1.0712x over previous
import functools

import jax
import jax.numpy as jnp
from jax import lax
from jax.experimental import pallas as pl
from jax.experimental.pallas import tpu as pltpu
from jax.experimental.pallas import tpu_sc as plsc

B, N, D, E = 4, 10000, 128, 160000
NT = B * N
ET = B * E
RPB = E // 128
RPW = 160
RPG = 8 * RPW
ROWS = B * RPG
HALF = 20480
NPAD = 2 * HALF
ACC_ROWS = NPAD + 128
CHALF = HALF + 16
K = NT // B // 2
F = 32

def _prep_body(ei4, srcg, dl, inv, sslab, dslab, cntv, tmpv, invv, stage):
    c = lax.axis_index("c")
    s = lax.axis_index("s")
    ones16 = jnp.ones((16,), jnp.float32)

    def zero_cnt(i, _):
        cntv[pl.ds(i * 16, 16)] = jnp.zeros((16,), jnp.float32)
        return 0

    lax.fori_loop(0, CHALF // 16, zero_cnt, 0)

    for half in range(2):
        row_base = (2 * s + half) * RPW
        b = row_base // RPG
        rib = row_base - b * RPG
        pltpu.sync_copy(ei4.at[b, 0, pl.ds(rib, RPW)], sslab)
        pltpu.sync_copy(ei4.at[b, 1, pl.ds(rib, RPW)], dslab)
        boff = b * N

        def row_body(q, _):
            realrow = rib + q < RPB
            for ci in range(8):
                sl = sslab[q, pl.ds(16 * ci, 16)]
                dg = dslab[q, pl.ds(16 * ci, 16)] + boff
                sg = jnp.where(realrow, sl + boff, 0)
                dgp = jnp.where(realrow, dg, NPAD)
                dloc = dg - c * HALF
                okc = realrow & (dloc >= 0) & (dloc < HALF)
                dcnt = jnp.where(okc, dloc, HALF)
                plsc.addupdate_scatter(cntv, [dcnt], ones16)
                sslab[q, pl.ds(16 * ci, 16)] = sg
                dslab[q, pl.ds(16 * ci, 16)] = dgp
            return 0

        lax.fori_loop(0, RPW, row_body, 0)

        @pl.when(c == 0)
        def _():
            pltpu.sync_copy(sslab, srcg.at[pl.ds(row_base, RPW)])
            pltpu.sync_copy(dslab, dl.at[pl.ds(row_base, RPW)])

    pltpu.sync_copy(cntv.at[pl.ds(0, HALF)], stage.at[s])
    plsc.subcore_barrier()
    for t in range(16):
        pltpu.sync_copy(stage.at[t, pl.ds(s * 1280, 1280)], tmpv.at[t])

    def red(jj, _):
        acc = tmpv[0, pl.ds(16 * jj, 16)]
        for t in range(1, 16):
            acc = acc + tmpv[t, pl.ds(16 * jj, 16)]
        invv[pl.ds(16 * jj, 16)] = 1.0 / jnp.maximum(acc, 1.0)
        return 0

    lax.fori_loop(0, 80, red, 0)
    pltpu.sync_copy(invv, inv.at[pl.ds(c * HALF + s * 1280, 1280)])


def _seg_body(t0, t1, t2, t3, srcg, dg, out, sidx, didx, stage, zbuf, acc,
              gsems, ssems, isems):
    c = lax.axis_index("c")
    s = lax.axis_index("s")

    def zz(i, _):
        for k in range(2):
            zbuf[i, pl.ds(16 * k, 16)] = jnp.zeros((16,), jnp.float32)
        return 0

    lax.fori_loop(0, 128, zz, 0)

    n_it = (ROWS // 2) // 16 // 8
    base = c * (ROWS // 2) + s * (8 * n_it)

    def idx_load(it, buf):
        r = base + 8 * jnp.minimum(it, n_it - 1)
        pltpu.async_copy(srcg.at[pl.ds(r, 8)], sidx.at[buf], isems.at[2 * buf])
        pltpu.async_copy(dg.at[pl.ds(r, 8)], didx.at[buf],
                         isems.at[2 * buf + 1])

    def idx_wait(buf):
        pltpu.make_async_copy(srcg.at[pl.ds(base, 8)], sidx.at[buf],
                              isems.at[2 * buf]).wait()
        pltpu.make_async_copy(dg.at[pl.ds(base, 8)], didx.at[buf],
                              isems.at[2 * buf + 1]).wait()

    for p, tbl in enumerate((t0, t1, t2, t3)):
        zb = s * (ACC_ROWS // 16)
        for k in range(20):
            pltpu.sync_copy(zbuf, acc.at[pl.ds(zb + 128 * k, 128)])
        pltpu.sync_copy(zbuf.at[pl.ds(0, ACC_ROWS // 16 - 2560)],
                        acc.at[pl.ds(zb + 2560, ACC_ROWS // 16 - 2560)])
        plsc.subcore_barrier()

        idx_load(0, 0)

        def i_body(i, _):
            for par in range(2):
                it = 2 * i + par
                @pl.when(it > 0)
                def _():
                    for j in range(8):
                        pltpu.make_async_copy(stage.at[pl.ds(128 * j, 128)],
                                              acc.at[didx.at[1 - par, j]],
                                              ssems.at[j]).wait()

                idx_load(it + 1, 1 - par)
                idx_wait(par)
                descs = [
                    pltpu.async_copy(tbl.at[sidx.at[par, j]],
                                     stage.at[pl.ds(128 * j, 128)],
                                     gsems.at[j])
                    for j in range(8)
                ]
                for j in range(8):
                    descs[j].wait()
                    pltpu.async_copy(stage.at[pl.ds(128 * j, 128)],
                                     acc.at[didx.at[par, j]], ssems.at[j],
                                     add=True)
            return 0

        lax.fori_loop(0, n_it // 2, i_body, 0)
        idx_wait(0)
        for j in range(8):
            pltpu.make_async_copy(stage.at[pl.ds(128 * j, 128)],
                                  acc.at[didx.at[1, j]], ssems.at[j]).wait()
        plsc.subcore_barrier()
        pltpu.sync_copy(acc.at[pl.ds(s * 2560, 2560)],
                        out.at[c, p, pl.ds(s * 2560, 2560)])
        plsc.subcore_barrier()


@functools.lru_cache(maxsize=None)
def _sc_kernels():
    mesh = plsc.VectorSubcoreMesh(core_axis_name="c", subcore_axis_name="s")
    prep = pl.kernel(
        _prep_body,
        out_type=(
            jax.ShapeDtypeStruct((ROWS, 128), jnp.int32),
            jax.ShapeDtypeStruct((ROWS, 128), jnp.int32),
            jax.ShapeDtypeStruct((NPAD,), jnp.float32),
        ),
        mesh=mesh,
        scratch_types=[
            pltpu.VMEM((RPW, 128), jnp.int32),
            pltpu.VMEM((RPW, 128), jnp.int32),
            pltpu.VMEM((CHALF,), jnp.float32),
            pltpu.VMEM((16, 1280), jnp.float32),
            pltpu.VMEM((1280,), jnp.float32),
            pltpu.VMEM_SHARED((16, HALF), jnp.float32),
        ],
        compiler_params=pltpu.CompilerParams(
            needs_layout_passes=False, use_tc_tiling_on_sc=False),
    )
    seg = pl.kernel(
        _seg_body,
        out_type=jax.ShapeDtypeStruct((2, 4, NPAD, F), jnp.float32),
        mesh=mesh,
        scratch_types=[
            pltpu.VMEM((2, 8, 128), jnp.int32),
            pltpu.VMEM((2, 8, 128), jnp.int32),
            pltpu.VMEM((1024, F), jnp.float32),
            pltpu.VMEM((128, F), jnp.float32),
            pltpu.VMEM_SHARED((ACC_ROWS, F), jnp.float32),
            pltpu.SemaphoreType.DMA((8,)),
            pltpu.SemaphoreType.DMA((8,)),
            pltpu.SemaphoreType.DMA((4,)),
        ],
        compiler_params=pltpu.CompilerParams(
            needs_layout_passes=False, use_tc_tiling_on_sc=False),
    )
    return prep, seg


def _layer_body(relu, agg, h0, h1, h2, h3, inv, Wl, Wr, bl, o0, o1, o2, o3):
    a = agg[...]
    ap = a[0] + a[1]
    mean = jnp.concatenate([ap[0], ap[1], ap[2], ap[3]], axis=1) * inv[...]
    x = jnp.concatenate([h0[...], h1[...], h2[...], h3[...]], axis=1)
    y = (jnp.dot(mean, Wl[...], preferred_element_type=jnp.float32)
         + jnp.dot(x, Wr[...], preferred_element_type=jnp.float32) + bl[...])
    if relu:
        y = jnp.maximum(y, 0.0)
    o0[...] = y[:, 0 * F:1 * F]
    o1[...] = y[:, 1 * F:2 * F]
    o2[...] = y[:, 2 * F:3 * F]
    o3[...] = y[:, 3 * F:4 * F]


def _layer(agg, hs, inv2, Wl, Wr, bl, relu):
    R = 2560
    grid = NPAD // R
    io = lambda i: (i, 0)
    w0 = lambda i: (0, 0)
    return pl.pallas_call(
        functools.partial(_layer_body, relu),
        grid=(grid,),
        in_specs=[
            pl.BlockSpec((2, 4, R, F), lambda i: (0, 0, i, 0)),
            pl.BlockSpec((R, F), io), pl.BlockSpec((R, F), io),
            pl.BlockSpec((R, F), io), pl.BlockSpec((R, F), io),
            pl.BlockSpec((R, 1), io),
            pl.BlockSpec((128, 128), w0), pl.BlockSpec((128, 128), w0),
            pl.BlockSpec((1, 128), w0),
        ],
        out_specs=[pl.BlockSpec((R, F), io)] * 4,
        out_shape=[jax.ShapeDtypeStruct((NPAD, F), jnp.float32)] * 4,
    )(agg, *hs, inv2, Wl, Wr, bl.reshape(1, 128))


def _pool_body(h0, h1, h2, h3, pw, pooled):
    x = jnp.concatenate([h0[...], h1[...], h2[...], h3[...]], axis=1)
    pwv = pw[...]
    pwn = pwv * lax.rsqrt(jnp.sum(pwv * pwv))
    sc = lax.dot_general(x, pwn, (((1,), (1,)), ((), ())),
                         preferred_element_type=jnp.float32)
    kb = lax.bitcast_convert_type(sc, jnp.int32)
    k = kb ^ ((kb >> 31) & jnp.int32(0x7FFFFFFF))
    ku = lax.bitcast_convert_type(k ^ jnp.int32(-2147483648), jnp.uint32)
    top = jnp.uint32(2147483648)

    def step(j, t):
        cand = t | lax.shift_right_logical(top, jnp.uint32(j))
        cnt = jnp.sum((ku >= cand).astype(jnp.int32))
        return jnp.where(cnt >= K, cand, t)

    t = lax.fori_loop(0, 32, step, jnp.uint32(0))
    w = jnp.tanh(sc) * (ku >= t).astype(jnp.float32)
    pr = lax.dot_general(w, x, (((0,), (0,)), ((), ())),
                         preferred_element_type=jnp.float32)
    pooled[pl.ds(pl.program_id(0), 1), :] = pr * (1.0 / K)


def _pool(hs, pool_w):
    return pl.pallas_call(
        _pool_body,
        grid=(B,),
        in_specs=[pl.BlockSpec((N, F), lambda b: (b, 0))] * 4
        + [pl.BlockSpec((1, 128), lambda b: (0, 0))],
        out_specs=pl.BlockSpec((B, 128), lambda b: (0, 0)),
        out_shape=jax.ShapeDtypeStruct((B, 128), jnp.float32),
    )(*hs, pool_w.reshape(1, 128))


def _mlp_body(p, W1, b1, W2, b2, o):
    h = jnp.maximum(
        jnp.dot(p[...], W1[...], preferred_element_type=jnp.float32) + b1[...],
        0.0)
    o[...] = jnp.dot(h, W2[...], preferred_element_type=jnp.float32) + b2[...]


def _mlp(p, W1, b1, W2, b2):
    fs = lambda *shape: pl.BlockSpec(shape, lambda: tuple(0 for _ in shape))
    return pl.pallas_call(
        _mlp_body,
        in_specs=[fs(B, 128), fs(128, 256), fs(1, 256), fs(256, 10), fs(1, 10)],
        out_specs=fs(B, 10),
        out_shape=jax.ShapeDtypeStruct((B, 10), jnp.float32),
    )(p, W1, b1.reshape(1, 256), W2, b2.reshape(1, 10))


def kernel(freq, edge_index, edge_weight, Wl0, bl0, Wr0, Wl1, bl1, Wr1,
           Wl2, bl2, Wr2, pool_w, W1, b1, W2, b2):
    ei4 = jnp.pad(edge_index.reshape(B, 2, RPB, 128),
                  ((0, 0), (0, 0), (0, RPG - RPB), (0, 0)))
    _prep, _seg = _sc_kernels()
    srcg, dgp, inv = _prep(ei4)
    inv2 = inv.reshape(NPAD, 1)
    x = freq.reshape(NT, D)
    xp = jnp.pad(x, ((0, NPAD - NT), (0, 0)))
    hs = [xp[:, i * F:(i + 1) * F] for i in range(4)]
    for i, (Wl, bl, Wr) in enumerate(((Wl0, bl0, Wr0), (Wl1, bl1, Wr1),
                                      (Wl2, bl2, Wr2))):
        agg = _seg(*hs, srcg, dgp)
        hs = _layer(agg, hs, inv2, Wl, Wr, bl, relu=(i < 2))
    pooled = _pool([h[:NT] for h in hs], pool_w)
    return _mlp(pooled, W1, b1, W2, b2)


# speedup vs baseline: 3.2892x; 1.0038x over previous
import functools

import jax
import jax.numpy as jnp
from jax import lax
from jax.experimental import pallas as pl
from jax.experimental.pallas import tpu as pltpu
from jax.experimental.pallas import tpu_sc as plsc

B, N, D, E = 4, 10000, 128, 160000
NT = B * N
ET = B * E
RPB = E // 128
RPW = 160
RPG = 8 * RPW
ROWS = B * RPG
HALF = 20480
NPAD = 2 * HALF
ACC_ROWS = NPAD + 128
CHALF = HALF + 16
K = NT // B // 2
F = 32

def _prep_body(ei4, srcg, dl, inv, sslab, dslab, cntv, tmpv, invv, stage):
    c = lax.axis_index("c")
    s = lax.axis_index("s")
    ones16 = jnp.ones((16,), jnp.float32)

    def zero_cnt(i, _):
        cntv[pl.ds(i * 16, 16)] = jnp.zeros((16,), jnp.float32)
        return 0

    lax.fori_loop(0, CHALF // 16, zero_cnt, 0)

    for half in range(2):
        row_base = (2 * s + half) * RPW
        b = row_base // RPG
        rib = row_base - b * RPG
        pltpu.sync_copy(ei4.at[b, 0, pl.ds(rib, RPW)], sslab)
        pltpu.sync_copy(ei4.at[b, 1, pl.ds(rib, RPW)], dslab)
        boff = b * N

        def row_body(q, _):
            realrow = rib + q < RPB
            for ci in range(8):
                sl = sslab[q, pl.ds(16 * ci, 16)]
                dg = dslab[q, pl.ds(16 * ci, 16)] + boff
                sg = jnp.where(realrow, sl + boff, 0)
                dgp = jnp.where(realrow, dg, NPAD)
                dloc = dg - c * HALF
                okc = realrow & (dloc >= 0) & (dloc < HALF)
                dcnt = jnp.where(okc, dloc, HALF)
                plsc.addupdate_scatter(cntv, [dcnt], ones16)
                sslab[q, pl.ds(16 * ci, 16)] = sg
                dslab[q, pl.ds(16 * ci, 16)] = dgp
            return 0

        lax.fori_loop(0, RPW, row_body, 0)

        @pl.when(c == 0)
        def _():
            pltpu.sync_copy(sslab, srcg.at[pl.ds(row_base, RPW)])
            pltpu.sync_copy(dslab, dl.at[pl.ds(row_base, RPW)])

    pltpu.sync_copy(cntv.at[pl.ds(0, HALF)], stage.at[s])
    plsc.subcore_barrier()
    for t in range(16):
        pltpu.sync_copy(stage.at[t, pl.ds(s * 1280, 1280)], tmpv.at[t])

    def red(jj, _):
        acc = tmpv[0, pl.ds(16 * jj, 16)]
        for t in range(1, 16):
            acc = acc + tmpv[t, pl.ds(16 * jj, 16)]
        invv[pl.ds(16 * jj, 16)] = 1.0 / jnp.maximum(acc, 1.0)
        return 0

    lax.fori_loop(0, 80, red, 0)
    pltpu.sync_copy(invv, inv.at[pl.ds(c * HALF + s * 1280, 1280)])


def _seg_body(t0, t1, t2, t3, srcg, dg, out, sidx, didx, stage, zbuf, acc,
              gsems, ssems, isems):
    c = lax.axis_index("c")
    s = lax.axis_index("s")

    def zz(i, _):
        for k in range(2):
            zbuf[i, pl.ds(16 * k, 16)] = jnp.zeros((16,), jnp.float32)
        return 0

    lax.fori_loop(0, 128, zz, 0)

    n_it = (ROWS // 2) // 16 // 8
    base = c * (ROWS // 2) + s * (8 * n_it)

    def idx_load(it, buf):
        r = base + 8 * jnp.minimum(it, n_it - 1)
        pltpu.async_copy(srcg.at[pl.ds(r, 8)], sidx.at[buf], isems.at[2 * buf])
        pltpu.async_copy(dg.at[pl.ds(r, 8)], didx.at[buf],
                         isems.at[2 * buf + 1])

    def idx_wait(buf):
        pltpu.make_async_copy(srcg.at[pl.ds(base, 8)], sidx.at[buf],
                              isems.at[2 * buf]).wait()
        pltpu.make_async_copy(dg.at[pl.ds(base, 8)], didx.at[buf],
                              isems.at[2 * buf + 1]).wait()

    for p, tbl in enumerate((t0, t1, t2, t3)):
        zb = s * (ACC_ROWS // 16)
        zd = [pltpu.async_copy(zbuf, acc.at[pl.ds(zb + 128 * k, 128)],
                               gsems.at[k % 8]) for k in range(20)]
        zd.append(pltpu.async_copy(
            zbuf.at[pl.ds(0, ACC_ROWS // 16 - 2560)],
            acc.at[pl.ds(zb + 2560, ACC_ROWS // 16 - 2560)], gsems.at[4]))
        for d in zd:
            d.wait()
        plsc.subcore_barrier()

        idx_load(0, 0)

        def i_body(i, _):
            for par in range(2):
                it = 2 * i + par
                @pl.when(it > 0)
                def _():
                    for j in range(8):
                        pltpu.make_async_copy(stage.at[pl.ds(128 * j, 128)],
                                              acc.at[didx.at[1 - par, j]],
                                              ssems.at[j]).wait()

                idx_load(it + 1, 1 - par)
                idx_wait(par)
                descs = [
                    pltpu.async_copy(tbl.at[sidx.at[par, j]],
                                     stage.at[pl.ds(128 * j, 128)],
                                     gsems.at[j])
                    for j in range(8)
                ]
                for j in range(8):
                    descs[j].wait()
                    pltpu.async_copy(stage.at[pl.ds(128 * j, 128)],
                                     acc.at[didx.at[par, j]], ssems.at[j],
                                     add=True)
            return 0

        lax.fori_loop(0, n_it // 2, i_body, 0)
        idx_wait(0)
        for j in range(8):
            pltpu.make_async_copy(stage.at[pl.ds(128 * j, 128)],
                                  acc.at[didx.at[1, j]], ssems.at[j]).wait()
        plsc.subcore_barrier()
        pltpu.sync_copy(acc.at[pl.ds(s * 2560, 2560)],
                        out.at[c, p, pl.ds(s * 2560, 2560)])
        plsc.subcore_barrier()


@functools.lru_cache(maxsize=None)
def _sc_kernels():
    mesh = plsc.VectorSubcoreMesh(core_axis_name="c", subcore_axis_name="s")
    prep = pl.kernel(
        _prep_body,
        out_type=(
            jax.ShapeDtypeStruct((ROWS, 128), jnp.int32),
            jax.ShapeDtypeStruct((ROWS, 128), jnp.int32),
            jax.ShapeDtypeStruct((NPAD,), jnp.float32),
        ),
        mesh=mesh,
        scratch_types=[
            pltpu.VMEM((RPW, 128), jnp.int32),
            pltpu.VMEM((RPW, 128), jnp.int32),
            pltpu.VMEM((CHALF,), jnp.float32),
            pltpu.VMEM((16, 1280), jnp.float32),
            pltpu.VMEM((1280,), jnp.float32),
            pltpu.VMEM_SHARED((16, HALF), jnp.float32),
        ],
        compiler_params=pltpu.CompilerParams(
            needs_layout_passes=False, use_tc_tiling_on_sc=False),
    )
    seg = pl.kernel(
        _seg_body,
        out_type=jax.ShapeDtypeStruct((2, 4, NPAD, F), jnp.float32),
        mesh=mesh,
        scratch_types=[
            pltpu.VMEM((2, 8, 128), jnp.int32),
            pltpu.VMEM((2, 8, 128), jnp.int32),
            pltpu.VMEM((1024, F), jnp.float32),
            pltpu.VMEM((128, F), jnp.float32),
            pltpu.VMEM_SHARED((ACC_ROWS, F), jnp.float32),
            pltpu.SemaphoreType.DMA((8,)),
            pltpu.SemaphoreType.DMA((8,)),
            pltpu.SemaphoreType.DMA((4,)),
        ],
        compiler_params=pltpu.CompilerParams(
            needs_layout_passes=False, use_tc_tiling_on_sc=False),
    )
    return prep, seg


def _layer_body(relu, agg, h0, h1, h2, h3, inv, Wl, Wr, bl, o0, o1, o2, o3):
    a = agg[...]
    ap = a[0] + a[1]
    mean = jnp.concatenate([ap[0], ap[1], ap[2], ap[3]], axis=1) * inv[...]
    x = jnp.concatenate([h0[...], h1[...], h2[...], h3[...]], axis=1)
    y = (jnp.dot(mean, Wl[...], preferred_element_type=jnp.float32)
         + jnp.dot(x, Wr[...], preferred_element_type=jnp.float32) + bl[...])
    if relu:
        y = jnp.maximum(y, 0.0)
    o0[...] = y[:, 0 * F:1 * F]
    o1[...] = y[:, 1 * F:2 * F]
    o2[...] = y[:, 2 * F:3 * F]
    o3[...] = y[:, 3 * F:4 * F]


def _layer(agg, hs, inv2, Wl, Wr, bl, relu):
    R = 2560
    grid = NPAD // R
    io = lambda i: (i, 0)
    w0 = lambda i: (0, 0)
    return pl.pallas_call(
        functools.partial(_layer_body, relu),
        grid=(grid,),
        in_specs=[
            pl.BlockSpec((2, 4, R, F), lambda i: (0, 0, i, 0)),
            pl.BlockSpec((R, F), io), pl.BlockSpec((R, F), io),
            pl.BlockSpec((R, F), io), pl.BlockSpec((R, F), io),
            pl.BlockSpec((R, 1), io),
            pl.BlockSpec((128, 128), w0), pl.BlockSpec((128, 128), w0),
            pl.BlockSpec((1, 128), w0),
        ],
        out_specs=[pl.BlockSpec((R, F), io)] * 4,
        out_shape=[jax.ShapeDtypeStruct((NPAD, F), jnp.float32)] * 4,
    )(agg, *hs, inv2, Wl, Wr, bl.reshape(1, 128))


def _pool_body(h0, h1, h2, h3, pw, pooled):
    x = jnp.concatenate([h0[...], h1[...], h2[...], h3[...]], axis=1)
    pwv = pw[...]
    pwn = pwv * lax.rsqrt(jnp.sum(pwv * pwv))
    sc = lax.dot_general(x, pwn, (((1,), (1,)), ((), ())),
                         preferred_element_type=jnp.float32)
    kb = lax.bitcast_convert_type(sc, jnp.int32)
    k = kb ^ ((kb >> 31) & jnp.int32(0x7FFFFFFF))
    ku = lax.bitcast_convert_type(k ^ jnp.int32(-2147483648), jnp.uint32)
    top = jnp.uint32(2147483648)

    def step(j, t):
        cand = t | lax.shift_right_logical(top, jnp.uint32(j))
        cnt = jnp.sum((ku >= cand).astype(jnp.int32))
        return jnp.where(cnt >= K, cand, t)

    t = lax.fori_loop(0, 32, step, jnp.uint32(0))
    w = jnp.tanh(sc) * (ku >= t).astype(jnp.float32)
    pr = lax.dot_general(w, x, (((0,), (0,)), ((), ())),
                         preferred_element_type=jnp.float32)
    pooled[pl.ds(pl.program_id(0), 1), :] = pr * (1.0 / K)


def _pool(hs, pool_w):
    return pl.pallas_call(
        _pool_body,
        grid=(B,),
        in_specs=[pl.BlockSpec((N, F), lambda b: (b, 0))] * 4
        + [pl.BlockSpec((1, 128), lambda b: (0, 0))],
        out_specs=pl.BlockSpec((B, 128), lambda b: (0, 0)),
        out_shape=jax.ShapeDtypeStruct((B, 128), jnp.float32),
    )(*hs, pool_w.reshape(1, 128))


def _mlp_body(p, W1, b1, W2, b2, o):
    h = jnp.maximum(
        jnp.dot(p[...], W1[...], preferred_element_type=jnp.float32) + b1[...],
        0.0)
    o[...] = jnp.dot(h, W2[...], preferred_element_type=jnp.float32) + b2[...]


def _mlp(p, W1, b1, W2, b2):
    fs = lambda *shape: pl.BlockSpec(shape, lambda: tuple(0 for _ in shape))
    return pl.pallas_call(
        _mlp_body,
        in_specs=[fs(B, 128), fs(128, 256), fs(1, 256), fs(256, 10), fs(1, 10)],
        out_specs=fs(B, 10),
        out_shape=jax.ShapeDtypeStruct((B, 10), jnp.float32),
    )(p, W1, b1.reshape(1, 256), W2, b2.reshape(1, 10))


def kernel(freq, edge_index, edge_weight, Wl0, bl0, Wr0, Wl1, bl1, Wr1,
           Wl2, bl2, Wr2, pool_w, W1, b1, W2, b2):
    ei4 = jnp.pad(edge_index.reshape(B, 2, RPB, 128),
                  ((0, 0), (0, 0), (0, RPG - RPB), (0, 0)))
    _prep, _seg = _sc_kernels()
    srcg, dgp, inv = _prep(ei4)
    inv2 = inv.reshape(NPAD, 1)
    x = freq.reshape(NT, D)
    xp = jnp.pad(x, ((0, NPAD - NT), (0, 0)))
    hs = [xp[:, i * F:(i + 1) * F] for i in range(4)]
    for i, (Wl, bl, Wr) in enumerate(((Wl0, bl0, Wr0), (Wl1, bl1, Wr1),
                                      (Wl2, bl2, Wr2))):
        agg = _seg(*hs, srcg, dgp)
        hs = _layer(agg, hs, inv2, Wl, Wr, bl, relu=(i < 2))
    pooled = _pool([h[:NT] for h in hs], pool_w)
    return _mlp(pooled, W1, b1, W2, b2)
